# baseline (speedup 1.0000x reference)
import functools

import jax
import jax.numpy as jnp
from jax import lax
from jax.experimental import pallas as pl
from jax.experimental.pallas import tpu as pltpu
from jax.experimental.pallas import tpu_sc as plsc

_B, _S, _H, _V = 4, 8192, 1024, 100000
_NW = 32
_TOK = _B * _S
_TPW = _TOK // _NW
_C = 32
_NCHUNK = _TPW // _C
_VPR = _H // 16


def _emb_kernel(ids_hbm, pt_hbm, wt_hbm, out_hbm, idx_v, rows_v, pos_v, sem):
    wid = lax.axis_index("s") * 2 + lax.axis_index("c")
    base = wid * _TPW
    s_base = lax.rem(base, _S)
    pltpu.sync_copy(ids_hbm.at[pl.ds(base, _TPW)], idx_v)

    def chunk_body(i, carry):
        off = i * _C
        gat = pltpu.async_copy(
            wt_hbm.at[idx_v.at[pl.ds(off, _C)]], rows_v, sem
        )
        pltpu.sync_copy(pt_hbm.at[pl.ds(s_base + off, _C)], pos_v)
        gat.wait()

        def row_body(r, c2):
            for k in range(_VPR):
                sl = pl.ds(k * 16, 16)
                plsc.addupdate(rows_v.at[r, sl], pos_v[r, sl])
            return c2

        lax.fori_loop(0, _C, row_body, 0)
        pltpu.sync_copy(rows_v, out_hbm.at[pl.ds(base + off, _C)])
        return carry

    lax.fori_loop(0, _NCHUNK, chunk_body, 0)


@jax.jit
def _run(ids_flat, word_table, pos_table):
    mesh = plsc.VectorSubcoreMesh(core_axis_name="c", subcore_axis_name="s")
    k = functools.partial(
        pl.kernel,
        mesh=mesh,
        out_type=jax.ShapeDtypeStruct((_TOK, _H), jnp.float32),
        scratch_types=[
            pltpu.VMEM((_TPW,), jnp.int32),
            pltpu.VMEM((_C, _H), jnp.float32),
            pltpu.VMEM((_C, _H), jnp.float32),
            pltpu.SemaphoreType.DMA,
        ],
    )(_emb_kernel)
    return k(ids_flat, pos_table, word_table)


def kernel(input_ids, word_table, pos_table):
    ids_flat = input_ids.reshape(_TOK).astype(jnp.int32)
    out = _run(ids_flat, word_table, pos_table)
    return out.reshape(_B, _S, _H)


# speedup vs baseline: 1.1348x; 1.1348x over previous
import functools

import jax
import jax.numpy as jnp
from jax import lax
from jax.experimental import pallas as pl
from jax.experimental.pallas import tpu as pltpu
from jax.experimental.pallas import tpu_sc as plsc

_B, _S, _H, _V = 4, 8192, 1024, 100000
_NW = 32
_TOK = _B * _S
_SPW = _S // _NW
_C = 32
_NSTRIPE = _SPW // _C
_NCHUNK = _NSTRIPE * _B
_VPR = _H // 16


def _add_pos(rows_ref, pos_ref):
    def row_body(r, c):
        for k in range(_VPR):
            sl = pl.ds(k * 16, 16)
            plsc.addupdate(rows_ref.at[r, sl], pos_ref[r, sl])
        return c

    lax.fori_loop(0, _C, row_body, 0)


def _emb_kernel(ids_hbm, pt_hbm, wt_hbm, out_hbm,
                idx_v, pos_v, rows0, rows1, g0, g1, s0, s1):
    wid = lax.axis_index("s") * 2 + lax.axis_index("c")
    s_base = wid * _SPW
    rows = (rows0, rows1)
    gsem = (g0, g1)
    ssem = (s0, s1)

    for b in range(_B):
        pltpu.sync_copy(
            ids_hbm.at[pl.ds(b * _S + s_base, _SPW)],
            idx_v.at[pl.ds(b * _SPW, _SPW)],
        )

    def gather_src(c):
        b = lax.rem(c, _B)
        j = lax.div(c, _B)
        return wt_hbm.at[idx_v.at[pl.ds(b * _SPW + j * _C, _C)]]

    def out_dst(c):
        b = lax.rem(c, _B)
        j = lax.div(c, _B)
        return out_hbm.at[pl.ds(b * _S + s_base + j * _C, _C)]

    pltpu.async_copy(gather_src(0), rows0, g0)
    pltpu.async_copy(gather_src(1), rows1, g1)

    def half_body(h, carry):
        for p in (0, 1):
            c = 2 * h + p
            rbuf = rows[p]
            pltpu.make_async_copy(gather_src(c), rbuf, gsem[p]).wait()
            @pl.when(lax.rem(c, _B) == 0)
            def _():
                j = lax.div(c, _B)
                pltpu.sync_copy(pt_hbm.at[pl.ds(s_base + j * _C, _C)], pos_v)

            _add_pos(rbuf, pos_v)
            pltpu.async_copy(rbuf, out_dst(c), ssem[p])
            @pl.when(c >= 1)
            def _():
                pltpu.make_async_copy(rows[1 - p], out_dst(c - 1), ssem[1 - p]).wait()

            @pl.when((c >= 1) & (c + 1 < _NCHUNK))
            def _():
                pltpu.async_copy(gather_src(c + 1), rows[1 - p], gsem[1 - p])
        return carry

    lax.fori_loop(0, _NCHUNK // 2, half_body, 0)
    pltpu.make_async_copy(rows1, out_dst(_NCHUNK - 1), ssem[1]).wait()


@jax.jit
def _run(ids_flat, word_table, pos_table):
    mesh = plsc.VectorSubcoreMesh(core_axis_name="c", subcore_axis_name="s")
    k = functools.partial(
        pl.kernel,
        mesh=mesh,
        out_type=jax.ShapeDtypeStruct((_TOK, _H), jnp.float32),
        scratch_types=[
            pltpu.VMEM((_B * _SPW,), jnp.int32),
            pltpu.VMEM((_C, _H), jnp.float32),
            pltpu.VMEM((_C, _H), jnp.float32),
            pltpu.VMEM((_C, _H), jnp.float32),
            pltpu.SemaphoreType.DMA,
            pltpu.SemaphoreType.DMA,
            pltpu.SemaphoreType.DMA,
            pltpu.SemaphoreType.DMA,
        ],
    )(_emb_kernel)
    return k(ids_flat, pos_table, word_table)


def kernel(input_ids, word_table, pos_table):
    ids_flat = input_ids.reshape(_TOK).astype(jnp.int32)
    out = _run(ids_flat, word_table, pos_table)
    return out.reshape(_B, _S, _H)


# speedup vs baseline: 1.9688x; 1.7350x over previous
import functools

import jax
import jax.numpy as jnp
from jax import lax
from jax.experimental import pallas as pl
from jax.experimental.pallas import tpu as pltpu
from jax.experimental.pallas import tpu_sc as plsc

_B, _S, _H, _V = 4, 8192, 1024, 100000
_NW = 32
_TOK = _B * _S
_SPW = _S // _NW
_C = 32
_NSTRIPE = _SPW // _C
_NCHUNK = _NSTRIPE * _B
_VPR = _H // 16


def _add_pos(rows_ref, pos_ref):
    _G = 8

    def row_body(r, c):
        for g in range(_VPR // _G):
            vals = [pos_ref[r, pl.ds((g * _G + u) * 16, 16)] for u in range(_G)]
            for u in range(_G):
                plsc.addupdate(rows_ref.at[r, pl.ds((g * _G + u) * 16, 16)], vals[u])
        return c

    lax.fori_loop(0, _C, row_body, 0)


def _emb_kernel(ids_hbm, pt_hbm, wt_hbm, out_hbm,
                idx_v, pos_v, rows0, rows1, g0, g1, s0, s1):
    wid = lax.axis_index("s") * 2 + lax.axis_index("c")
    s_base = wid * _SPW
    rows = (rows0, rows1)
    gsem = (g0, g1)
    ssem = (s0, s1)

    for b in range(_B):
        pltpu.sync_copy(
            ids_hbm.at[pl.ds(b * _S + s_base, _SPW)],
            idx_v.at[pl.ds(b * _SPW, _SPW)],
        )

    def gather_src(c):
        b = lax.rem(c, _B)
        j = lax.div(c, _B)
        return wt_hbm.at[idx_v.at[pl.ds(b * _SPW + j * _C, _C)]]

    def out_dst(c):
        b = lax.rem(c, _B)
        j = lax.div(c, _B)
        return out_hbm.at[pl.ds(b * _S + s_base + j * _C, _C)]

    pltpu.async_copy(gather_src(0), rows0, g0)
    pltpu.async_copy(gather_src(1), rows1, g1)

    def half_body(h, carry):
        for p in (0, 1):
            c = 2 * h + p
            rbuf = rows[p]
            pltpu.make_async_copy(gather_src(c), rbuf, gsem[p]).wait()
            @pl.when(lax.rem(c, _B) == 0)
            def _():
                j = lax.div(c, _B)
                pltpu.sync_copy(pt_hbm.at[pl.ds(s_base + j * _C, _C)], pos_v)

            _add_pos(rbuf, pos_v)
            pltpu.async_copy(rbuf, out_dst(c), ssem[p])
            @pl.when(c >= 1)
            def _():
                pltpu.make_async_copy(rows[1 - p], out_dst(c - 1), ssem[1 - p]).wait()

            @pl.when((c >= 1) & (c + 1 < _NCHUNK))
            def _():
                pltpu.async_copy(gather_src(c + 1), rows[1 - p], gsem[1 - p])
        return carry

    lax.fori_loop(0, _NCHUNK // 2, half_body, 0)
    pltpu.make_async_copy(rows1, out_dst(_NCHUNK - 1), ssem[1]).wait()


@jax.jit
def _run(ids_flat, word_table, pos_table):
    mesh = plsc.VectorSubcoreMesh(core_axis_name="c", subcore_axis_name="s")
    k = functools.partial(
        pl.kernel,
        mesh=mesh,
        out_type=jax.ShapeDtypeStruct((_TOK, _H), jnp.float32),
        scratch_types=[
            pltpu.VMEM((_B * _SPW,), jnp.int32),
            pltpu.VMEM((_C, _H), jnp.float32),
            pltpu.VMEM((_C, _H), jnp.float32),
            pltpu.VMEM((_C, _H), jnp.float32),
            pltpu.SemaphoreType.DMA,
            pltpu.SemaphoreType.DMA,
            pltpu.SemaphoreType.DMA,
            pltpu.SemaphoreType.DMA,
        ],
    )(_emb_kernel)
    return k(ids_flat, pos_table, word_table)


def kernel(input_ids, word_table, pos_table):
    ids_flat = input_ids.reshape(_TOK).astype(jnp.int32)
    out = _run(ids_flat, word_table, pos_table)
    return out.reshape(_B, _S, _H)


# speedup vs baseline: 2.3500x; 1.1936x over previous
import functools

import jax
import jax.numpy as jnp
from jax import lax
from jax.experimental import pallas as pl
from jax.experimental.pallas import tpu as pltpu
from jax.experimental.pallas import tpu_sc as plsc

_B, _S, _H, _V = 4, 8192, 1024, 100000
_NW = 32
_TOK = _B * _S
_SPW = _S // _NW
_C = 32
_NSTRIPE = _SPW // _C
_NCHUNK = _NSTRIPE * _B
_VPR = _H // 16


def _add_pos(rows_ref, pos_ref):
    _G = 8
    _NG = _VPR // _G

    def _loads(r, g):
        return [pos_ref[r, pl.ds((g * _G + u) * 16, 16)] for u in range(_G)]

    def _stores(r, g, vals):
        for u in range(_G):
            plsc.addupdate(rows_ref.at[r, pl.ds((g * _G + u) * 16, 16)], vals[u])

    def row_body(r, c):
        vals = _loads(r, 0)
        for g in range(1, _NG):
            nxt = _loads(r, g)
            _stores(r, g - 1, vals)
            vals = nxt
        _stores(r, _NG - 1, vals)
        return c

    lax.fori_loop(0, _C, row_body, 0)


def _emb_kernel(ids_hbm, pt_hbm, wt_hbm, out_hbm,
                idx_v, pos_v, rows0, rows1, g0, g1, s0, s1):
    wid = lax.axis_index("s") * 2 + lax.axis_index("c")
    s_base = wid * _SPW
    rows = (rows0, rows1)
    gsem = (g0, g1)
    ssem = (s0, s1)

    for b in range(_B):
        pltpu.sync_copy(
            ids_hbm.at[pl.ds(b * _S + s_base, _SPW)],
            idx_v.at[pl.ds(b * _SPW, _SPW)],
        )

    def gather_src(c):
        b = lax.rem(c, _B)
        j = lax.div(c, _B)
        return wt_hbm.at[idx_v.at[pl.ds(b * _SPW + j * _C, _C)]]

    def out_dst(c):
        b = lax.rem(c, _B)
        j = lax.div(c, _B)
        return out_hbm.at[pl.ds(b * _S + s_base + j * _C, _C)]

    pltpu.async_copy(gather_src(0), rows0, g0)
    pltpu.async_copy(gather_src(1), rows1, g1)

    def half_body(h, carry):
        for p in (0, 1):
            c = 2 * h + p
            rbuf = rows[p]
            pltpu.make_async_copy(gather_src(c), rbuf, gsem[p]).wait()
            @pl.when(c >= 1)
            def _():
                pltpu.make_async_copy(rows[1 - p], out_dst(c - 1), ssem[1 - p]).wait()

            @pl.when((c >= 1) & (c + 1 < _NCHUNK))
            def _():
                pltpu.async_copy(gather_src(c + 1), rows[1 - p], gsem[1 - p])

            @pl.when(lax.rem(c, _B) == 0)
            def _():
                j = lax.div(c, _B)
                pltpu.sync_copy(pt_hbm.at[pl.ds(s_base + j * _C, _C)], pos_v)

            _add_pos(rbuf, pos_v)
            pltpu.async_copy(rbuf, out_dst(c), ssem[p])
        return carry

    lax.fori_loop(0, _NCHUNK // 2, half_body, 0)
    pltpu.make_async_copy(rows1, out_dst(_NCHUNK - 1), ssem[1]).wait()


@jax.jit
def _run(ids_flat, word_table, pos_table):
    mesh = plsc.VectorSubcoreMesh(core_axis_name="c", subcore_axis_name="s")
    k = functools.partial(
        pl.kernel,
        mesh=mesh,
        out_type=jax.ShapeDtypeStruct((_TOK, _H), jnp.float32),
        scratch_types=[
            pltpu.VMEM((_B * _SPW,), jnp.int32),
            pltpu.VMEM((_C, _H), jnp.float32),
            pltpu.VMEM((_C, _H), jnp.float32),
            pltpu.VMEM((_C, _H), jnp.float32),
            pltpu.SemaphoreType.DMA,
            pltpu.SemaphoreType.DMA,
            pltpu.SemaphoreType.DMA,
            pltpu.SemaphoreType.DMA,
        ],
    )(_emb_kernel)
    return k(ids_flat, pos_table, word_table)


def kernel(input_ids, word_table, pos_table):
    ids_flat = input_ids.reshape(_TOK).astype(jnp.int32)
    out = _run(ids_flat, word_table, pos_table)
    return out.reshape(_B, _S, _H)


# speedup vs baseline: 2.7202x; 1.1576x over previous
import functools

import jax
import jax.numpy as jnp
from jax import lax
from jax.experimental import pallas as pl
from jax.experimental.pallas import tpu as pltpu
from jax.experimental.pallas import tpu_sc as plsc

_B, _S, _H, _V = 4, 8192, 1024, 100000
_NW = 32
_TOK = _B * _S
_SPW = _S // _NW
_C = 16
_NSTRIPE = _SPW // _C
_NCHUNK = _NSTRIPE * _B
_VPR = _H // 16
_NBUF = 4


def _add_pos(rows_ref, pos_ref):
    _G = 8
    _NG = _VPR // _G

    def _loads(r, g):
        return [pos_ref[r, pl.ds((g * _G + u) * 16, 16)] for u in range(_G)]

    def _stores(r, g, vals):
        for u in range(_G):
            plsc.addupdate(rows_ref.at[r, pl.ds((g * _G + u) * 16, 16)], vals[u])

    def row_body(r, c):
        vals = _loads(r, 0)
        for g in range(1, _NG):
            nxt = _loads(r, g)
            _stores(r, g - 1, vals)
            vals = nxt
        _stores(r, _NG - 1, vals)
        return c

    lax.fori_loop(0, _C, row_body, 0)


def _emb_kernel(ids_hbm, pt_hbm, wt_hbm, out_hbm,
                idx_v, pos_v, rows0, rows1, rows2, rows3,
                g0, g1, g2, g3, s0, s1, s2, s3):
    wid = lax.axis_index("s") * 2 + lax.axis_index("c")
    s_base = wid * _SPW
    rows = (rows0, rows1, rows2, rows3)
    gsem = (g0, g1, g2, g3)
    ssem = (s0, s1, s2, s3)

    for b in range(_B):
        pltpu.sync_copy(
            ids_hbm.at[pl.ds(b * _S + s_base, _SPW)],
            idx_v.at[pl.ds(b * _SPW, _SPW)],
        )

    def gather_src(c):
        b = lax.rem(c, _B)
        j = lax.div(c, _B)
        return wt_hbm.at[idx_v.at[pl.ds(b * _SPW + j * _C, _C)]]

    def out_dst(c):
        b = lax.rem(c, _B)
        j = lax.div(c, _B)
        return out_hbm.at[pl.ds(b * _S + s_base + j * _C, _C)]

    pltpu.async_copy(gather_src(0), rows0, g0)
    pltpu.async_copy(gather_src(1), rows1, g1)

    def quad_body(h, carry):
        for q in range(_NBUF):
            c = _NBUF * h + q
            rbuf = rows[q]
            pltpu.make_async_copy(gather_src(c), rbuf, gsem[q]).wait()
            qn = (q + 2) % _NBUF

            @pl.when(c >= 2)
            def _():
                pltpu.make_async_copy(rows[qn], out_dst(c - 2), ssem[qn]).wait()

            @pl.when(c + 2 < _NCHUNK)
            def _():
                pltpu.async_copy(gather_src(c + 2), rows[qn], gsem[qn])

            @pl.when(lax.rem(c, _B) == 0)
            def _():
                j = lax.div(c, _B)
                pltpu.sync_copy(pt_hbm.at[pl.ds(s_base + j * _C, _C)], pos_v)

            _add_pos(rbuf, pos_v)
            pltpu.async_copy(rbuf, out_dst(c), ssem[q])
        return carry

    lax.fori_loop(0, _NCHUNK // _NBUF, quad_body, 0)
    pltpu.make_async_copy(rows[(_NCHUNK - 2) % _NBUF],
                          out_dst(_NCHUNK - 2),
                          ssem[(_NCHUNK - 2) % _NBUF]).wait()
    pltpu.make_async_copy(rows[(_NCHUNK - 1) % _NBUF],
                          out_dst(_NCHUNK - 1),
                          ssem[(_NCHUNK - 1) % _NBUF]).wait()


@jax.jit
def _run(ids_flat, word_table, pos_table):
    mesh = plsc.VectorSubcoreMesh(core_axis_name="c", subcore_axis_name="s")
    k = functools.partial(
        pl.kernel,
        mesh=mesh,
        out_type=jax.ShapeDtypeStruct((_TOK, _H), jnp.float32),
        scratch_types=[
            pltpu.VMEM((_B * _SPW,), jnp.int32),
            pltpu.VMEM((_C, _H), jnp.float32),
        ] + [pltpu.VMEM((_C, _H), jnp.float32)] * _NBUF
          + [pltpu.SemaphoreType.DMA] * (2 * _NBUF),
    )(_emb_kernel)
    return k(ids_flat, pos_table, word_table)


def kernel(input_ids, word_table, pos_table):
    ids_flat = input_ids.reshape(_TOK).astype(jnp.int32)
    out = _run(ids_flat, word_table, pos_table)
    return out.reshape(_B, _S, _H)


# speedup vs baseline: 2.9118x; 1.0704x over previous
import functools

import jax
import jax.numpy as jnp
from jax import lax
from jax.experimental import pallas as pl
from jax.experimental.pallas import tpu as pltpu
from jax.experimental.pallas import tpu_sc as plsc

_B, _S, _H, _V = 4, 8192, 1024, 100000
_NW = 32
_TOK = _B * _S
_SPW = _S // _NW
_P = 8
_NG = _SPW // _P
_GR = _B * _P
_VPR = _H // 16
_NBUF = 3


def _add_pos(rows_ref, pos_ref):
    _G = 8

    def row_body(r, c):
        for g in range(_VPR // _G):
            vals = [pos_ref[r, pl.ds((g * _G + u) * 16, 16)] for u in range(_G)]
            for b in range(_B):
                for u in range(_G):
                    plsc.addupdate(
                        rows_ref.at[b * _P + r, pl.ds((g * _G + u) * 16, 16)],
                        vals[u],
                    )
        return c

    lax.fori_loop(0, _P, row_body, 0)


def _emb_kernel(ids_hbm, pt_hbm, wt_hbm, out_hbm,
                idx_lin, pos0, pos1, rows0, rows1, rows2,
                g0, g1, g2, s0, s1, s2, p0, p1):
    wid = lax.axis_index("s") * 2 + lax.axis_index("c")
    s_base = wid * _SPW
    rows = (rows0, rows1, rows2)
    gsem = (g0, g1, g2)
    ssem = (s0, s1, s2)
    pos = (pos0, pos1)
    psem = (p0, p1)

    for b in range(_B):
        pltpu.sync_copy(
            ids_hbm.at[pl.ds(b * _S + s_base, _SPW)],
            idx_lin.at[pl.ds(b * _SPW, _SPW)],
        )

    def gather_pair(g, b, q):
        return (wt_hbm.at[idx_lin.at[pl.ds(b * _SPW + g * _P, _P)]],
                rows[q].at[pl.ds(b * _P, _P)])

    def issue_gather(g, q):
        for b in range(_B):
            src, dst = gather_pair(g, b, q)
            pltpu.async_copy(src, dst, gsem[q])

    def wait_gather(g, q):
        for b in range(_B):
            src, dst = gather_pair(g, b, q)
            pltpu.make_async_copy(src, dst, gsem[q]).wait()

    def store_pair(g, b, q):
        return (rows[q].at[pl.ds(b * _P, _P)],
                out_hbm.at[pl.ds(b * _S + s_base + g * _P, _P)])

    def pos_src(g):
        return pt_hbm.at[pl.ds(s_base + g * _P, _P)]

    pltpu.async_copy(pos_src(0), pos0, p0)
    issue_gather(0, 0)
    issue_gather(1, 1)

    def body(g, q, pp):
        rbuf = rows[q]
        wait_gather(g, q)
        @pl.when(g + 1 < _NG)
        def _():
            pltpu.async_copy(pos_src(g + 1), pos[1 - pp], psem[1 - pp])

        qn = (q + 2) % _NBUF

        @pl.when(g >= 1)
        def _():
            for b in range(_B):
                src, dst = store_pair(g - 1, b, qn)
                pltpu.make_async_copy(src, dst, ssem[qn]).wait()

        @pl.when(g + 2 < _NG)
        def _():
            issue_gather(g + 2, qn)

        pltpu.make_async_copy(pos_src(g), pos[pp], psem[pp]).wait()
        _add_pos(rbuf, pos[pp])
        for b in range(_B):
            src, dst = store_pair(g, b, q)
            pltpu.async_copy(src, dst, ssem[q])

    def six_body(h, carry):
        for u in range(6):
            body(6 * h + u, u % _NBUF, u % 2)
        return carry

    lax.fori_loop(0, _NG // 6, six_body, 0)
    for g in range(_NG - (_NG % 6), _NG):
        body(g, g % _NBUF, g % 2)
    gl = _NG - 1
    for b in range(_B):
        src, dst = store_pair(gl, b, gl % _NBUF)
        pltpu.make_async_copy(src, dst, ssem[gl % _NBUF]).wait()


@jax.jit
def _run(ids_flat, word_table, pos_table):
    mesh = plsc.VectorSubcoreMesh(core_axis_name="c", subcore_axis_name="s")
    k = functools.partial(
        pl.kernel,
        mesh=mesh,
        out_type=jax.ShapeDtypeStruct((_TOK, _H), jnp.float32),
        scratch_types=[
            pltpu.VMEM((_B * _SPW,), jnp.int32),
            pltpu.VMEM((_P, _H), jnp.float32),
            pltpu.VMEM((_P, _H), jnp.float32),
        ] + [pltpu.VMEM((_GR, _H), jnp.float32)] * _NBUF
          + [pltpu.SemaphoreType.DMA] * (2 * _NBUF + 2),
    )(_emb_kernel)
    return k(ids_flat, pos_table, word_table)


def kernel(input_ids, word_table, pos_table):
    ids_flat = input_ids.reshape(_TOK).astype(jnp.int32)
    out = _run(ids_flat, word_table, pos_table)
    return out.reshape(_B, _S, _H)


# speedup vs baseline: 2.9504x; 1.0132x over previous
import functools

import jax
import jax.numpy as jnp
from jax import lax
from jax.experimental import pallas as pl
from jax.experimental.pallas import tpu as pltpu
from jax.experimental.pallas import tpu_sc as plsc

_B, _S, _H, _V = 4, 8192, 1024, 100000
_NW = 32
_TOK = _B * _S
_SPW = _S // _NW
_P = 8
_NG = _SPW // _P
_GR = _B * _P
_VPR = _H // 16
_NBUF = 3


def _add_pos(rows_ref, pos_ref):
    _G = 8

    def row_body(r, c):
        for g in range(_VPR // _G):
            vals = [pos_ref[r, pl.ds((g * _G + u) * 16, 16)] for u in range(_G)]
            for b in range(_B):
                for u in range(_G):
                    plsc.addupdate(
                        rows_ref.at[b * _P + r, pl.ds((g * _G + u) * 16, 16)],
                        vals[u],
                    )
        return c

    lax.fori_loop(0, _P, row_body, 0)


def _emb_kernel(ids_hbm, pt_hbm, wt_hbm, out_hbm,
                idx_lin, pos0, pos1, rows0, rows1, rows2,
                g0, g1, g2, s0, s1, s2, p0, p1):
    wid = lax.axis_index("s") * 2 + lax.axis_index("c")
    s_base = wid * _SPW
    rows = (rows0, rows1, rows2)
    gsem = (g0, g1, g2)
    ssem = (s0, s1, s2)
    pos = (pos0, pos1)
    psem = (p0, p1)

    pltpu.sync_copy(ids_hbm.at[pl.ds(wid * (_B * _SPW), _B * _SPW)], idx_lin)

    def issue_gather(g, q):
        pltpu.async_copy(
            wt_hbm.at[idx_lin.at[pl.ds(g * _GR, _GR)]], rows[q], gsem[q]
        )

    def wait_gather(g, q):
        pltpu.make_async_copy(
            wt_hbm.at[idx_lin.at[pl.ds(g * _GR, _GR)]], rows[q], gsem[q]
        ).wait()

    def store_pair(g, b, q):
        return (rows[q].at[pl.ds(b * _P, _P)],
                out_hbm.at[pl.ds(b * _S + s_base + g * _P, _P)])

    def pos_src(g):
        return pt_hbm.at[pl.ds(s_base + g * _P, _P)]

    pltpu.async_copy(pos_src(0), pos0, p0)
    issue_gather(0, 0)
    issue_gather(1, 1)

    def body(g, q, pp):
        rbuf = rows[q]
        wait_gather(g, q)
        @pl.when(g + 1 < _NG)
        def _():
            pltpu.async_copy(pos_src(g + 1), pos[1 - pp], psem[1 - pp])

        qn = (q + 2) % _NBUF

        @pl.when(g >= 1)
        def _():
            for b in range(_B):
                src, dst = store_pair(g - 1, b, qn)
                pltpu.make_async_copy(src, dst, ssem[qn]).wait()

        @pl.when(g + 2 < _NG)
        def _():
            issue_gather(g + 2, qn)

        pltpu.make_async_copy(pos_src(g), pos[pp], psem[pp]).wait()
        _add_pos(rbuf, pos[pp])
        for b in range(_B):
            src, dst = store_pair(g, b, q)
            pltpu.async_copy(src, dst, ssem[q])

    def six_body(h, carry):
        for u in range(6):
            body(6 * h + u, u % _NBUF, u % 2)
        return carry

    lax.fori_loop(0, _NG // 6, six_body, 0)
    for g in range(_NG - (_NG % 6), _NG):
        body(g, g % _NBUF, g % 2)
    gl = _NG - 1
    for b in range(_B):
        src, dst = store_pair(gl, b, gl % _NBUF)
        pltpu.make_async_copy(src, dst, ssem[gl % _NBUF]).wait()


@jax.jit
def _run(ids_flat, word_table, pos_table):
    mesh = plsc.VectorSubcoreMesh(core_axis_name="c", subcore_axis_name="s")
    k = functools.partial(
        pl.kernel,
        mesh=mesh,
        out_type=jax.ShapeDtypeStruct((_TOK, _H), jnp.float32),
        scratch_types=[
            pltpu.VMEM((_B * _SPW,), jnp.int32),
            pltpu.VMEM((_P, _H), jnp.float32),
            pltpu.VMEM((_P, _H), jnp.float32),
        ] + [pltpu.VMEM((_GR, _H), jnp.float32)] * _NBUF
          + [pltpu.SemaphoreType.DMA] * (2 * _NBUF + 2),
    )(_emb_kernel)
    return k(ids_flat, pos_table, word_table)


def kernel(input_ids, word_table, pos_table):
    ids_g = jnp.transpose(
        input_ids.astype(jnp.int32).reshape(_B, _NW, _NG, _P), (1, 2, 0, 3)
    ).reshape(_TOK)
    out = _run(ids_g, word_table, pos_table)
    return out.reshape(_B, _S, _H)


# speedup vs baseline: 3.1404x; 1.0644x over previous
import functools

import jax
import jax.numpy as jnp
from jax import lax
from jax.experimental import pallas as pl
from jax.experimental.pallas import tpu as pltpu
from jax.experimental.pallas import tpu_sc as plsc

_B, _S, _H, _V = 4, 8192, 1024, 100000
_NW = 32
_TOK = _B * _S
_SPW = _S // _NW
_P = 8
_NG = _SPW // _P
_GR = _B * _P
_VPR = _H // 16
_NBUF = 3


def _add_pos(rows_ref, pos_ref):
    _G = 8

    def row_body(r, c):
        for g in range(_VPR // _G):
            vals = [pos_ref[r, pl.ds((g * _G + u) * 16, 16)] for u in range(_G)]
            for b in range(_B):
                for u in range(_G):
                    plsc.addupdate(
                        rows_ref.at[b * _P + r, pl.ds((g * _G + u) * 16, 16)],
                        vals[u],
                    )
        return c

    @functools.partial(plsc.parallel_loop, 0, _P)
    def _(r):
        row_body(r, 0)


def _emb_kernel(ids_hbm, pt_hbm, wt_hbm, out_hbm,
                idx_lin, pos0, pos1, rows0, rows1, rows2,
                g0, g1, g2, s0, s1, s2, p0, p1):
    wid = lax.axis_index("s") * 2 + lax.axis_index("c")
    s_base = wid * _SPW
    rows = (rows0, rows1, rows2)
    gsem = (g0, g1, g2)
    ssem = (s0, s1, s2)
    pos = (pos0, pos1)
    psem = (p0, p1)

    pltpu.sync_copy(ids_hbm.at[pl.ds(wid * (_B * _SPW), _B * _SPW)], idx_lin)

    def issue_gather(g, q):
        pltpu.async_copy(
            wt_hbm.at[idx_lin.at[pl.ds(g * _GR, _GR)]], rows[q], gsem[q]
        )

    def wait_gather(g, q):
        pltpu.make_async_copy(
            wt_hbm.at[idx_lin.at[pl.ds(g * _GR, _GR)]], rows[q], gsem[q]
        ).wait()

    def store_pair(g, b, q):
        return (rows[q].at[pl.ds(b * _P, _P)],
                out_hbm.at[pl.ds(b * _S + s_base + g * _P, _P)])

    def pos_src(g):
        return pt_hbm.at[pl.ds(s_base + g * _P, _P)]

    pltpu.async_copy(pos_src(0), pos0, p0)
    issue_gather(0, 0)
    issue_gather(1, 1)

    def body(g, q, pp):
        rbuf = rows[q]
        wait_gather(g, q)
        @pl.when(g + 1 < _NG)
        def _():
            pltpu.async_copy(pos_src(g + 1), pos[1 - pp], psem[1 - pp])

        qn = (q + 2) % _NBUF

        @pl.when(g >= 1)
        def _():
            for b in range(_B):
                src, dst = store_pair(g - 1, b, qn)
                pltpu.make_async_copy(src, dst, ssem[qn]).wait()

        @pl.when(g + 2 < _NG)
        def _():
            issue_gather(g + 2, qn)

        pltpu.make_async_copy(pos_src(g), pos[pp], psem[pp]).wait()
        _add_pos(rbuf, pos[pp])
        for b in range(_B):
            src, dst = store_pair(g, b, q)
            pltpu.async_copy(src, dst, ssem[q])

    def six_body(h, carry):
        for u in range(6):
            body(6 * h + u, u % _NBUF, u % 2)
        return carry

    lax.fori_loop(0, _NG // 6, six_body, 0)
    for g in range(_NG - (_NG % 6), _NG):
        body(g, g % _NBUF, g % 2)
    gl = _NG - 1
    for b in range(_B):
        src, dst = store_pair(gl, b, gl % _NBUF)
        pltpu.make_async_copy(src, dst, ssem[gl % _NBUF]).wait()


@jax.jit
def _run(ids_flat, word_table, pos_table):
    mesh = plsc.VectorSubcoreMesh(core_axis_name="c", subcore_axis_name="s")
    k = functools.partial(
        pl.kernel,
        mesh=mesh,
        out_type=jax.ShapeDtypeStruct((_TOK, _H), jnp.float32),
        scratch_types=[
            pltpu.VMEM((_B * _SPW,), jnp.int32),
            pltpu.VMEM((_P, _H), jnp.float32),
            pltpu.VMEM((_P, _H), jnp.float32),
        ] + [pltpu.VMEM((_GR, _H), jnp.float32)] * _NBUF
          + [pltpu.SemaphoreType.DMA] * (2 * _NBUF + 2),
    )(_emb_kernel)
    return k(ids_flat, pos_table, word_table)


def kernel(input_ids, word_table, pos_table):
    ids_g = jnp.transpose(
        input_ids.astype(jnp.int32).reshape(_B, _NW, _NG, _P), (1, 2, 0, 3)
    ).reshape(_TOK)
    out = _run(ids_g, word_table, pos_table)
    return out.reshape(_B, _S, _H)
